# Initial kernel scaffold; baseline (speedup 1.0000x reference)
import functools

import jax
import jax.numpy as jnp
from jax import lax
from jax.experimental import pallas as pl
from jax.experimental.pallas import tpu as pltpu
from jax.experimental.pallas import tpu_sc as plsc

EMBEDDING_DIM = 32
BATCH = 16384
HIST_LEN = 200
B_TOTAL = BATCH * HIST_LEN

NUM_CORES = 2
NUM_SUBCORES = 16
NUM_WORKERS = NUM_CORES * NUM_SUBCORES
B_PER_WORKER = B_TOTAL // NUM_WORKERS
CHUNK = 2048
N_ITERS = B_PER_WORKER // CHUNK

_mesh = plsc.VectorSubcoreMesh(core_axis_name="c", subcore_axis_name="s")


@functools.partial(
    pl.kernel,
    mesh=_mesh,
    out_type=jax.ShapeDtypeStruct((B_TOTAL, EMBEDDING_DIM), jnp.float32),
    scratch_types=[
        pltpu.VMEM((CHUNK,), jnp.int32),
        pltpu.VMEM((CHUNK, EMBEDDING_DIM), jnp.float32),
        pltpu.SemaphoreType.DMA,
    ],
)
def _gather_kernel(idx_hbm, table_hbm, out_hbm, idx_v, rows_v, sem):
    wid = lax.axis_index("s") * NUM_CORES + lax.axis_index("c")
    base = wid * B_PER_WORKER

    def body(i, carry):
        off = base + i * CHUNK
        pltpu.sync_copy(idx_hbm.at[pl.ds(off, CHUNK)], idx_v)
        pltpu.async_copy(table_hbm.at[idx_v], rows_v, sem).wait()
        pltpu.sync_copy(rows_v, out_hbm.at[pl.ds(off, CHUNK)])
        return carry

    lax.fori_loop(0, N_ITERS, body, 0)


def kernel(token_ids, embedding_table):
    flat = token_ids.reshape(-1).astype(jnp.int32)
    out = _gather_kernel(flat, embedding_table)
    return out.reshape(token_ids.shape + (EMBEDDING_DIM,))


# speedup vs baseline: 4.9482x; 4.9482x over previous
import functools

import jax
import jax.numpy as jnp
from jax import lax
from jax.experimental import pallas as pl
from jax.experimental.pallas import tpu as pltpu
from jax.experimental.pallas import tpu_sc as plsc

EMBEDDING_DIM = 32
BATCH = 16384
HIST_LEN = 200
B_TOTAL = BATCH * HIST_LEN

NUM_CORES = 2
NUM_SUBCORES = 16
NUM_WORKERS = NUM_CORES * NUM_SUBCORES
B_PER_WORKER = B_TOTAL // NUM_WORKERS
CHUNK = 2048
N_ITERS = B_PER_WORKER // CHUNK

_mesh = plsc.VectorSubcoreMesh(core_axis_name="c", subcore_axis_name="s")


@functools.partial(
    pl.kernel,
    mesh=_mesh,
    out_type=jax.ShapeDtypeStruct((B_TOTAL, EMBEDDING_DIM), jnp.float32),
    scratch_types=[
        pltpu.VMEM((CHUNK,), jnp.int32),
        pltpu.VMEM((CHUNK, EMBEDDING_DIM), jnp.float32),
        pltpu.SemaphoreType.DMA,
    ],
    compiler_params=pltpu.CompilerParams(use_tc_tiling_on_sc=False),
)
def _gather_kernel(idx_hbm, table_hbm, out_hbm, idx_v, rows_v, sem):
    wid = lax.axis_index("s") * NUM_CORES + lax.axis_index("c")
    base = wid * B_PER_WORKER

    def body(i, carry):
        off = base + i * CHUNK
        pltpu.sync_copy(idx_hbm.at[pl.ds(off, CHUNK)], idx_v)
        pltpu.async_copy(table_hbm.at[idx_v], rows_v, sem).wait()
        pltpu.sync_copy(rows_v, out_hbm.at[pl.ds(off, CHUNK)])
        return carry

    lax.fori_loop(0, N_ITERS, body, 0)


def kernel(token_ids, embedding_table):
    flat = token_ids.reshape(-1).astype(jnp.int32)
    out = _gather_kernel(flat, embedding_table)
    return out.reshape(token_ids.shape + (EMBEDDING_DIM,))


# speedup vs baseline: 5.0508x; 1.0207x over previous
import functools

import jax
import jax.numpy as jnp
from jax import lax
from jax.experimental import pallas as pl
from jax.experimental.pallas import tpu as pltpu
from jax.experimental.pallas import tpu_sc as plsc

EMBEDDING_DIM = 32
BATCH = 16384
HIST_LEN = 200
B_TOTAL = BATCH * HIST_LEN

NUM_CORES = 2
NUM_SUBCORES = 16
NUM_WORKERS = NUM_CORES * NUM_SUBCORES
B_PER_WORKER = B_TOTAL // NUM_WORKERS

CHUNK = 512
NBUF = 4
N_CHUNKS = B_PER_WORKER // CHUNK
N_GROUPS = N_CHUNKS // NBUF

_mesh = plsc.VectorSubcoreMesh(core_axis_name="c", subcore_axis_name="s")

_scratch = [pltpu.VMEM((NBUF, CHUNK), jnp.int32),
            pltpu.VMEM((NBUF, CHUNK, EMBEDDING_DIM), jnp.float32)]
_scratch += [pltpu.SemaphoreType.DMA] * (3 * NBUF)


@functools.partial(
    pl.kernel,
    mesh=_mesh,
    out_type=jax.ShapeDtypeStruct((B_TOTAL, EMBEDDING_DIM), jnp.float32),
    scratch_types=_scratch,
    compiler_params=pltpu.CompilerParams(use_tc_tiling_on_sc=False),
)
def _gather_kernel(idx_hbm, table_hbm, out_hbm, idx_v, rows_v, *sems):
    idx_sems = sems[0:NBUF]
    gat_sems = sems[NBUF:2 * NBUF]
    out_sems = sems[2 * NBUF:3 * NBUF]

    wid = lax.axis_index("s") * NUM_CORES + lax.axis_index("c")
    base = wid * B_PER_WORKER
    last = jnp.int32(N_CHUNKS - 1)

    def start_idx(i, b):
        off = base + jnp.minimum(i, last) * CHUNK
        pltpu.async_copy(idx_hbm.at[pl.ds(off, CHUNK)], idx_v.at[b],
                         idx_sems[b])

    def wait_idx(b):
        pltpu.make_async_copy(idx_hbm.at[pl.ds(base, CHUNK)], idx_v.at[b],
                              idx_sems[b]).wait()

    def start_gather(b):
        pltpu.async_copy(table_hbm.at[idx_v.at[b]], rows_v.at[b], gat_sems[b])

    def wait_gather(b):
        pltpu.make_async_copy(table_hbm.at[idx_v.at[b]], rows_v.at[b],
                              gat_sems[b]).wait()

    def start_out(i, b):
        off = base + i * CHUNK
        pltpu.async_copy(rows_v.at[b], out_hbm.at[pl.ds(off, CHUNK)],
                         out_sems[b])

    def wait_out(b):
        pltpu.make_async_copy(rows_v.at[b], out_hbm.at[pl.ds(base, CHUNK)],
                              out_sems[b]).wait()

    for b in range(NBUF):
        start_idx(jnp.int32(b), b)
    for b in range(NBUF):
        wait_idx(b)
        start_gather(b)
        if b > 0:
            wait_gather(b - 1)
            start_out(jnp.int32(b - 1), b - 1)
            start_idx(jnp.int32(b - 1 + NBUF), b - 1)

    def body(g, carry):
        i0 = g * NBUF
        for b in range(NBUF):
            i = i0 + b
            wait_idx(b)
            wait_out(b)
            start_gather(b)
            pb = (b - 1) % NBUF
            wait_gather(pb)
            start_out(i - 1, pb)
            start_idx(i - 1 + NBUF, pb)
        return carry

    lax.fori_loop(1, N_GROUPS, body, jnp.int32(0))

    wait_gather(NBUF - 1)
    start_out(jnp.int32(N_CHUNKS - 1), NBUF - 1)
    for b in range(NBUF - 1):
        wait_idx(b)
    for b in range(NBUF):
        wait_out(b)


def kernel(token_ids, embedding_table):
    flat = token_ids.reshape(-1).astype(jnp.int32)
    out = _gather_kernel(flat, embedding_table)
    return out.reshape(token_ids.shape + (EMBEDDING_DIM,))
